# Initial kernel scaffold; baseline (speedup 1.0000x reference)
import jax
import jax.numpy as jnp
from jax.experimental import pallas as pl


def kernel(x, edge_index, batch, Wl1, Wr1, b1, p1, Wl2, Wr2, b2, p2, Wl3, Wr3, b3, p3, W1, bl1, W2, bl2, W3, bl3):
    raise NotImplementedError("write your pallas kernel here")


# speedup vs baseline: 8.6488x; 8.6488x over previous
import functools

import jax
import jax.numpy as jnp
from jax import lax
from jax.experimental import pallas as pl
from jax.experimental.pallas import tpu as pltpu
from jax.experimental.pallas import tpu_sc as plsc

N = 10000
NP = 10240
E = 320000
F = 128
T_OUT = 10

NC = 2
NS = 16
NW = NC * NS
EPW = E // NW
C = 80
NCHUNK = EPW // C
ZR = 128
RPT = NP // NS

BLK = 512
NBLK = NP // BLK


def _sc_agg_body(xs_hbm, alive_hbm, src_hbm, dst_hbm, agg_hbm, deg_hbm,
                 src_v, dst_v, vals_v, rows_v, z_v, zd_v, agg_sh, deg_sh, sem):
    cid = lax.axis_index("c")
    sid = lax.axis_index("s")

    def zrow(r, carry):
        for j in range(F // 16):
            z_v[r, pl.ds(j * 16, 16)] = jnp.zeros((16,), jnp.float32)
        return carry
    lax.fori_loop(0, ZR, zrow, 0)

    def zdrow(r, carry):
        zd_v[pl.ds(r * 16, 16)] = jnp.zeros((16,), jnp.float32)
        return carry
    lax.fori_loop(0, RPT // 16, zdrow, 0)

    r0 = sid * RPT
    for bkt in range(RPT // ZR):
        pltpu.sync_copy(z_v, agg_sh.at[pl.ds(r0 + bkt * ZR, ZR)])
    pltpu.sync_copy(zd_v, deg_sh.at[pl.ds(r0, RPT)])
    plsc.subcore_barrier()

    wid = sid * NC + cid
    ebase = wid * EPW

    def chunk(i, carry):
        base = ebase + i * C
        pltpu.sync_copy(src_hbm.at[pl.ds(base, C)], src_v)
        pltpu.sync_copy(dst_hbm.at[pl.ds(base, C)], dst_v)
        pltpu.async_copy(xs_hbm.at[src_v], rows_v, sem).wait()
        pltpu.sync_copy(rows_v, agg_sh.at[dst_v], add=True)
        pltpu.async_copy(alive_hbm.at[src_v], vals_v, sem).wait()
        pltpu.sync_copy(vals_v, deg_sh.at[dst_v], add=True)
        return carry
    lax.fori_loop(0, NCHUNK, chunk, 0)
    plsc.subcore_barrier()

    pltpu.sync_copy(agg_sh.at[pl.ds(r0, RPT)],
                    agg_hbm.at[pl.ds(cid * NP + r0, RPT)])
    pltpu.sync_copy(deg_sh.at[pl.ds(r0, RPT)],
                    deg_hbm.at[pl.ds(cid * NP + r0, RPT)])


@functools.cache
def _get_sc_agg():
    return pl.kernel(
        _sc_agg_body,
        out_type=[
            jax.ShapeDtypeStruct((2 * NP, F), jnp.float32),
            jax.ShapeDtypeStruct((2 * NP,), jnp.float32),
        ],
        mesh=plsc.VectorSubcoreMesh(core_axis_name="c", subcore_axis_name="s",
                                    num_cores=NC, num_subcores=NS),
        scratch_types=[
            pltpu.VMEM((C,), jnp.int32),
            pltpu.VMEM((C,), jnp.int32),
            pltpu.VMEM((C,), jnp.float32),
            pltpu.VMEM((C, F), jnp.float32),
            pltpu.VMEM((ZR, F), jnp.float32),
            pltpu.VMEM((RPT,), jnp.float32),
            pltpu.VMEM_SHARED((NP, F), jnp.float32),
            pltpu.VMEM_SHARED((NP,), jnp.float32),
            pltpu.SemaphoreType.DMA,
        ],
        name="sc_edge_agg",
    )


def _tc1_body(agg0_ref, agg1_ref, deg0_ref, deg1_ref, xs_ref, alive_ref,
              wl_ref, wr_ref, b_ref, pn_ref, h_ref, s_ref, key_ref):
    a = agg0_ref[...] + agg1_ref[...]
    deg = deg0_ref[...] + deg1_ref[...]
    mean = a / jnp.maximum(deg, 1.0)
    x = xs_ref[...]
    alive = alive_ref[...]
    h = jnp.dot(mean, wl_ref[...], preferred_element_type=jnp.float32)
    h = h + jnp.dot(x, wr_ref[...], preferred_element_type=jnp.float32)
    h = jnp.maximum(h + b_ref[...], 0.0) * alive
    s = jnp.tanh(jnp.dot(h, pn_ref[...], preferred_element_type=jnp.float32))
    h_ref[...] = h
    s_ref[...] = s
    u = lax.bitcast_convert_type(s, jnp.uint32)
    key = jnp.where(u & jnp.uint32(0x80000000) != jnp.uint32(0),
                    ~u, u | jnp.uint32(0x80000000))
    key_ref[...] = jnp.where(alive > 0.0, key, jnp.uint32(0))


_tc1 = pl.pallas_call(
    _tc1_body,
    grid=(NBLK,),
    in_specs=[
        pl.BlockSpec((BLK, F), lambda i: (i, 0)),
        pl.BlockSpec((BLK, F), lambda i: (i + NBLK, 0)),
        pl.BlockSpec((BLK, 1), lambda i: (i, 0)),
        pl.BlockSpec((BLK, 1), lambda i: (i + NBLK, 0)),
        pl.BlockSpec((BLK, F), lambda i: (i, 0)),
        pl.BlockSpec((BLK, 1), lambda i: (i, 0)),
        pl.BlockSpec((F, F), lambda i: (0, 0)),
        pl.BlockSpec((F, F), lambda i: (0, 0)),
        pl.BlockSpec((1, F), lambda i: (0, 0)),
        pl.BlockSpec((F, 1), lambda i: (0, 0)),
    ],
    out_specs=[
        pl.BlockSpec((BLK, F), lambda i: (i, 0)),
        pl.BlockSpec((BLK, 1), lambda i: (i, 0)),
        pl.BlockSpec((BLK, 1), lambda i: (i, 0)),
    ],
    out_shape=[
        jax.ShapeDtypeStruct((NP, F), jnp.float32),
        jax.ShapeDtypeStruct((NP, 1), jnp.float32),
        jax.ShapeDtypeStruct((NP, 1), jnp.uint32),
    ],
)


def _sel_body(key_ref, keep_ref, *, k_out):
    keys = key_ref[...].reshape(NP // 128, 128)
    idx = (lax.broadcasted_iota(jnp.int32, (NP // 128, 128), 0) * 128
           + lax.broadcasted_iota(jnp.int32, (NP // 128, 128), 1))

    def tbody(j, t):
        bit = jnp.left_shift(jnp.uint32(1), jnp.uint32(31) - j.astype(jnp.uint32))
        cand = t | bit
        cnt = jnp.sum((keys >= cand).astype(jnp.int32))
        return jnp.where(cnt >= k_out, cand, t)
    t = lax.fori_loop(0, 32, tbody, jnp.uint32(0))

    cnt_gt = jnp.sum((keys > t).astype(jnp.int32))
    need = k_out - cnt_gt
    eq = keys == t

    def mbody(j, m):
        cand = m | jnp.left_shift(jnp.int32(1), jnp.int32(13) - j)
        f = jnp.sum((eq & (idx < cand)).astype(jnp.int32))
        return jnp.where(f < need, cand, m)
    m = lax.fori_loop(0, 14, mbody, jnp.int32(0))

    keep = (keys > t) | (eq & (idx <= m) & (need > 0))
    keep_ref[...] = keep.astype(jnp.float32).reshape(NP, 1)


def _make_sel(k_out):
    return pl.pallas_call(
        functools.partial(_sel_body, k_out=k_out),
        out_shape=jax.ShapeDtypeStruct((NP, 1), jnp.float32),
    )


def _pack_body(h_ref, s_ref, keep_ref, xs_ref, g_ref):
    i = pl.program_id(0)
    h = h_ref[...]
    s = s_ref[...]
    keep = keep_ref[...]
    xn = h * s * keep
    xs_ref[...] = xn
    blk_max = jnp.max(jnp.where(keep > 0.0, xn, jnp.float32(-3.4e38)),
                      axis=0, keepdims=True)
    blk_sum = jnp.sum(xn, axis=0, keepdims=True)

    @pl.when(i == 0)
    def _():
        g_ref[...] = jnp.concatenate(
            [jnp.full((1, F), -3.4e38, jnp.float32),
             jnp.zeros((1, F), jnp.float32)], axis=1)
    cur = g_ref[...]
    g_ref[...] = jnp.concatenate(
        [jnp.maximum(cur[:, :F], blk_max), cur[:, F:] + blk_sum], axis=1)


_pack = pl.pallas_call(
    _pack_body,
    grid=(NBLK,),
    in_specs=[
        pl.BlockSpec((BLK, F), lambda i: (i, 0)),
        pl.BlockSpec((BLK, 1), lambda i: (i, 0)),
        pl.BlockSpec((BLK, 1), lambda i: (i, 0)),
    ],
    out_specs=[
        pl.BlockSpec((BLK, F), lambda i: (i, 0)),
        pl.BlockSpec((1, 2 * F), lambda i: (0, 0)),
    ],
    out_shape=[
        jax.ShapeDtypeStruct((NP, F), jnp.float32),
        jax.ShapeDtypeStruct((1, 2 * F), jnp.float32),
    ],
)


def _mlp_body(g1_ref, g2_ref, g3_ref, w1_ref, b1_ref, w2_ref, b2_ref,
              w3_ref, b3_ref, o_ref, *, k_list):
    def gz(g, k):
        return jnp.concatenate([g[:, :F], g[:, F:] / jnp.float32(k)], axis=1)
    z = (gz(g1_ref[...], k_list[0]) + gz(g2_ref[...], k_list[1])
         + gz(g3_ref[...], k_list[2]))
    z = jnp.maximum(jnp.dot(z, w1_ref[...], preferred_element_type=jnp.float32)
                    + b1_ref[...], 0.0)
    z = jnp.maximum(jnp.dot(z, w2_ref[...], preferred_element_type=jnp.float32)
                    + b2_ref[...], 0.0)
    o_ref[...] = (jnp.dot(z, w3_ref[...], preferred_element_type=jnp.float32)
                  + b3_ref[...])


def _make_mlp(k_list):
    return pl.pallas_call(
        functools.partial(_mlp_body, k_list=k_list),
        out_shape=jax.ShapeDtypeStruct((1, T_OUT), jnp.float32),
    )


_K_SEQ = [8000, 6400, 5120]
_SEL = [_make_sel(k) for k in _K_SEQ]
_MLP = _make_mlp(_K_SEQ)


def kernel(x, edge_index, batch, Wl1, Wr1, b1, p1, Wl2, Wr2, b2, p2,
           Wl3, Wr3, b3, p3, W1, bl1, W2, bl2, W3, bl3):
    src = edge_index[0]
    dst = edge_index[1]
    xs = jnp.zeros((NP, F), jnp.float32).at[:N].set(x)
    alive = jnp.zeros((NP,), jnp.float32).at[:N].set(1.0)

    layer_params = [(Wl1, Wr1, b1, p1), (Wl2, Wr2, b2, p2), (Wl3, Wr3, b3, p3)]
    g_out = []
    for l in range(3):
        Wl, Wr, b, p = layer_params[l]
        pn = (p / (jnp.linalg.norm(p) + 1e-16)).reshape(F, 1)
        agg, deg = _get_sc_agg()(xs, alive, src, dst)
        h, s, key = _tc1(agg, agg, deg.reshape(2 * NP, 1), deg.reshape(2 * NP, 1),
                         xs, alive.reshape(NP, 1), Wl, Wr, b.reshape(1, F), pn)
        keep = _SEL[l](key)
        xs, g = _pack(h, s, keep)
        alive = keep.reshape(NP)
        g_out.append(g)

    return _MLP(g_out[0], g_out[1], g_out[2],
                W1, bl1.reshape(1, -1), W2, bl2.reshape(1, -1),
                W3, bl3.reshape(1, -1))


# speedup vs baseline: 17.0683x; 1.9735x over previous
import functools

import jax
import jax.numpy as jnp
from jax import lax
from jax.experimental import pallas as pl
from jax.experimental.pallas import tpu as pltpu
from jax.experimental.pallas import tpu_sc as plsc

N = 10000
NP = 10240
E = 320000
F = 128
T_OUT = 10

NC = 2
NS = 16
NW = NC * NS
EPW = E // NW
C = 80
NCHUNK = EPW // C
ZR = 128
RPT = NP // NS

BLK = 512
NBLK = NP // BLK


def _sc_agg_body(xs_hbm, alive_hbm, src_hbm, dst_hbm, agg_hbm, deg_hbm,
                 srcT, dstT, db0, db1, rows0, rows1, vals0, vals1,
                 agg_sh, deg_sh, sg0, sg1, sv0, sv1, ssc0, ssc1, svs0, svs1):
    cid = lax.axis_index("c")
    sid = lax.axis_index("s")
    rows = [rows0, rows1]
    vals = [vals0, vals1]
    db = [db0, db1]
    sg = [sg0, sg1]
    sv = [sv0, sv1]
    ssc = [ssc0, ssc1]
    svs = [svs0, svs1]

    wid = sid * NC + cid
    pltpu.sync_copy(src_hbm.at[pl.ds(wid * EPW, EPW)], srcT)
    pltpu.sync_copy(dst_hbm.at[pl.ds(wid * EPW, EPW)], dstT)

    def zrow(r, carry):
        for j in range(F // 16):
            rows0[r, pl.ds(j * 16, 16)] = jnp.zeros((16,), jnp.float32)
        return carry
    lax.fori_loop(0, C, zrow, 0)
    for j in range(C // 16):
        vals0[pl.ds(j * 16, 16)] = jnp.zeros((16,), jnp.float32)

    r0 = sid * RPT
    for bkt in range(RPT // C):
        pltpu.sync_copy(rows0, agg_sh.at[pl.ds(r0 + bkt * C, C)])
        pltpu.sync_copy(vals0, deg_sh.at[pl.ds(r0 + bkt * C, C)])
    plsc.subcore_barrier()

    def gather(i, b):
        sl = srcT.at[pl.ds(i * C, C)]
        for j in range(C // 16):
            db[b][pl.ds(j * 16, 16)] = dstT[pl.ds(i * C + j * 16, 16)]
        pltpu.async_copy(xs_hbm.at[sl], rows[b], sg[b])
        pltpu.async_copy(alive_hbm.at[sl], vals[b], sv[b])

    def wait_gather(b):
        pltpu.make_async_copy(xs_hbm.at[pl.ds(0, C)], rows[b], sg[b]).wait()
        pltpu.make_async_copy(alive_hbm.at[pl.ds(0, C)], vals[b], sv[b]).wait()

    def scatter(b):
        pltpu.async_copy(rows[b], agg_sh.at[db[b]], ssc[b], add=True)
        pltpu.async_copy(vals[b], deg_sh.at[db[b]], svs[b], add=True)

    def wait_scatter(b):
        pltpu.make_async_copy(xs_hbm.at[pl.ds(0, C)], rows[b], ssc[b]).wait()
        pltpu.make_async_copy(alive_hbm.at[pl.ds(0, C)], vals[b], svs[b]).wait()

    gather(0, 0)
    gather(1, 1)
    wait_gather(0)
    scatter(0)

    def pair(g, carry):
        for b0 in range(2):
            i = 2 * g + 1 + b0
            b = (1 + b0) & 1
            wait_scatter(1 - b)
            gather(i + 1, 1 - b)
            wait_gather(b)
            scatter(b)
        return carry
    lax.fori_loop(0, (NCHUNK - 3) // 2, pair, 0)
    wait_scatter(0)
    gather(NCHUNK - 1, 0)
    wait_gather(1)
    scatter(1)
    wait_gather(0)
    scatter(0)
    wait_scatter(1)
    wait_scatter(0)
    plsc.subcore_barrier()

    pltpu.sync_copy(agg_sh.at[pl.ds(r0, RPT)],
                    agg_hbm.at[pl.ds(cid * NP + r0, RPT)])
    pltpu.sync_copy(deg_sh.at[pl.ds(r0, RPT)],
                    deg_hbm.at[pl.ds(cid * NP + r0, RPT)])


@functools.cache
def _get_sc_agg():
    return pl.kernel(
        _sc_agg_body,
        out_type=[
            jax.ShapeDtypeStruct((2 * NP, F), jnp.float32),
            jax.ShapeDtypeStruct((2 * NP,), jnp.float32),
        ],
        mesh=plsc.VectorSubcoreMesh(core_axis_name="c", subcore_axis_name="s",
                                    num_cores=NC, num_subcores=NS),
        scratch_types=[
            pltpu.VMEM((EPW,), jnp.int32),
            pltpu.VMEM((EPW,), jnp.int32),
            pltpu.VMEM((C,), jnp.int32),
            pltpu.VMEM((C,), jnp.int32),
            pltpu.VMEM((C, F), jnp.float32),
            pltpu.VMEM((C, F), jnp.float32),
            pltpu.VMEM((C,), jnp.float32),
            pltpu.VMEM((C,), jnp.float32),
            pltpu.VMEM_SHARED((NP, F), jnp.float32),
            pltpu.VMEM_SHARED((NP,), jnp.float32),
        ] + [pltpu.SemaphoreType.DMA] * 8,
        name="sc_edge_agg",
    )


def _tc1_body(agg0_ref, agg1_ref, deg0_ref, deg1_ref, xs_ref, alive_ref,
              wl_ref, wr_ref, b_ref, pn_ref, h_ref, s_ref, key_ref):
    a = agg0_ref[...] + agg1_ref[...]
    deg = deg0_ref[...] + deg1_ref[...]
    mean = a / jnp.maximum(deg, 1.0)
    x = xs_ref[...]
    alive = alive_ref[...]
    h = jnp.dot(mean, wl_ref[...], preferred_element_type=jnp.float32)
    h = h + jnp.dot(x, wr_ref[...], preferred_element_type=jnp.float32)
    h = jnp.maximum(h + b_ref[...], 0.0) * alive
    s = jnp.tanh(jnp.dot(h, pn_ref[...], preferred_element_type=jnp.float32))
    h_ref[...] = h
    s_ref[...] = s
    u = lax.bitcast_convert_type(s, jnp.uint32)
    key = jnp.where(u & jnp.uint32(0x80000000) != jnp.uint32(0),
                    ~u, u | jnp.uint32(0x80000000))
    key_ref[...] = jnp.where(alive > 0.0, key, jnp.uint32(0))


_tc1 = pl.pallas_call(
    _tc1_body,
    grid=(NBLK,),
    in_specs=[
        pl.BlockSpec((BLK, F), lambda i: (i, 0)),
        pl.BlockSpec((BLK, F), lambda i: (i + NBLK, 0)),
        pl.BlockSpec((BLK, 1), lambda i: (i, 0)),
        pl.BlockSpec((BLK, 1), lambda i: (i + NBLK, 0)),
        pl.BlockSpec((BLK, F), lambda i: (i, 0)),
        pl.BlockSpec((BLK, 1), lambda i: (i, 0)),
        pl.BlockSpec((F, F), lambda i: (0, 0)),
        pl.BlockSpec((F, F), lambda i: (0, 0)),
        pl.BlockSpec((1, F), lambda i: (0, 0)),
        pl.BlockSpec((F, 1), lambda i: (0, 0)),
    ],
    out_specs=[
        pl.BlockSpec((BLK, F), lambda i: (i, 0)),
        pl.BlockSpec((BLK, 1), lambda i: (i, 0)),
        pl.BlockSpec((BLK, 1), lambda i: (i, 0)),
    ],
    out_shape=[
        jax.ShapeDtypeStruct((NP, F), jnp.float32),
        jax.ShapeDtypeStruct((NP, 1), jnp.float32),
        jax.ShapeDtypeStruct((NP, 1), jnp.uint32),
    ],
)


def _sel_body(key_ref, keep_ref, *, k_out):
    keys = key_ref[...].reshape(NP // 128, 128)
    idx = (lax.broadcasted_iota(jnp.int32, (NP // 128, 128), 0) * 128
           + lax.broadcasted_iota(jnp.int32, (NP // 128, 128), 1))

    def tbody(j, t):
        bit = jnp.left_shift(jnp.uint32(1), jnp.uint32(31) - j.astype(jnp.uint32))
        cand = t | bit
        cnt = jnp.sum((keys >= cand).astype(jnp.int32))
        return jnp.where(cnt >= k_out, cand, t)
    t = lax.fori_loop(0, 32, tbody, jnp.uint32(0))

    cnt_gt = jnp.sum((keys > t).astype(jnp.int32))
    need = k_out - cnt_gt
    eq = keys == t

    def mbody(j, m):
        cand = m | jnp.left_shift(jnp.int32(1), jnp.int32(13) - j)
        f = jnp.sum((eq & (idx < cand)).astype(jnp.int32))
        return jnp.where(f < need, cand, m)
    m = lax.fori_loop(0, 14, mbody, jnp.int32(0))

    keep = (keys > t) | (eq & (idx <= m) & (need > 0))
    keep_ref[...] = keep.astype(jnp.float32).reshape(NP, 1)


def _make_sel(k_out):
    return pl.pallas_call(
        functools.partial(_sel_body, k_out=k_out),
        out_shape=jax.ShapeDtypeStruct((NP, 1), jnp.float32),
    )


def _pack_body(h_ref, s_ref, keep_ref, xs_ref, g_ref):
    i = pl.program_id(0)
    h = h_ref[...]
    s = s_ref[...]
    keep = keep_ref[...]
    xn = h * s * keep
    xs_ref[...] = xn
    blk_max = jnp.max(jnp.where(keep > 0.0, xn, jnp.float32(-3.4e38)),
                      axis=0, keepdims=True)
    blk_sum = jnp.sum(xn, axis=0, keepdims=True)

    @pl.when(i == 0)
    def _():
        g_ref[...] = jnp.concatenate(
            [jnp.full((1, F), -3.4e38, jnp.float32),
             jnp.zeros((1, F), jnp.float32)], axis=1)
    cur = g_ref[...]
    g_ref[...] = jnp.concatenate(
        [jnp.maximum(cur[:, :F], blk_max), cur[:, F:] + blk_sum], axis=1)


_pack = pl.pallas_call(
    _pack_body,
    grid=(NBLK,),
    in_specs=[
        pl.BlockSpec((BLK, F), lambda i: (i, 0)),
        pl.BlockSpec((BLK, 1), lambda i: (i, 0)),
        pl.BlockSpec((BLK, 1), lambda i: (i, 0)),
    ],
    out_specs=[
        pl.BlockSpec((BLK, F), lambda i: (i, 0)),
        pl.BlockSpec((1, 2 * F), lambda i: (0, 0)),
    ],
    out_shape=[
        jax.ShapeDtypeStruct((NP, F), jnp.float32),
        jax.ShapeDtypeStruct((1, 2 * F), jnp.float32),
    ],
)


def _mlp_body(g1_ref, g2_ref, g3_ref, w1_ref, b1_ref, w2_ref, b2_ref,
              w3_ref, b3_ref, o_ref, *, k_list):
    def gz(g, k):
        return jnp.concatenate([g[:, :F], g[:, F:] / jnp.float32(k)], axis=1)
    z = (gz(g1_ref[...], k_list[0]) + gz(g2_ref[...], k_list[1])
         + gz(g3_ref[...], k_list[2]))
    z = jnp.maximum(jnp.dot(z, w1_ref[...], preferred_element_type=jnp.float32)
                    + b1_ref[...], 0.0)
    z = jnp.maximum(jnp.dot(z, w2_ref[...], preferred_element_type=jnp.float32)
                    + b2_ref[...], 0.0)
    o_ref[...] = (jnp.dot(z, w3_ref[...], preferred_element_type=jnp.float32)
                  + b3_ref[...])


def _make_mlp(k_list):
    return pl.pallas_call(
        functools.partial(_mlp_body, k_list=k_list),
        out_shape=jax.ShapeDtypeStruct((1, T_OUT), jnp.float32),
    )


_K_SEQ = [8000, 6400, 5120]
_SEL = [_make_sel(k) for k in _K_SEQ]
_MLP = _make_mlp(_K_SEQ)


def kernel(x, edge_index, batch, Wl1, Wr1, b1, p1, Wl2, Wr2, b2, p2,
           Wl3, Wr3, b3, p3, W1, bl1, W2, bl2, W3, bl3):
    src = edge_index[0]
    dst = edge_index[1]
    xs = jnp.zeros((NP, F), jnp.float32).at[:N].set(x)
    alive = jnp.zeros((NP,), jnp.float32).at[:N].set(1.0)

    layer_params = [(Wl1, Wr1, b1, p1), (Wl2, Wr2, b2, p2), (Wl3, Wr3, b3, p3)]
    g_out = []
    for l in range(3):
        Wl, Wr, b, p = layer_params[l]
        pn = (p / (jnp.linalg.norm(p) + 1e-16)).reshape(F, 1)
        agg, deg = _get_sc_agg()(xs, alive, src, dst)
        h, s, key = _tc1(agg, agg, deg.reshape(2 * NP, 1), deg.reshape(2 * NP, 1),
                         xs, alive.reshape(NP, 1), Wl, Wr, b.reshape(1, F), pn)
        keep = _SEL[l](key)
        xs, g = _pack(h, s, keep)
        alive = keep.reshape(NP)
        g_out.append(g)

    return _MLP(g_out[0], g_out[1], g_out[2],
                W1, bl1.reshape(1, -1), W2, bl2.reshape(1, -1),
                W3, bl3.reshape(1, -1))


# speedup vs baseline: 24.7194x; 1.4483x over previous
import functools

import jax
import jax.numpy as jnp
from jax import lax
from jax.experimental import pallas as pl
from jax.experimental.pallas import tpu as pltpu
from jax.experimental.pallas import tpu_sc as plsc

N = 10000
NP = 10240
E = 320000
F = 128
T_OUT = 10

NC = 2
NS = 16
NW = NC * NS
EPW = E // NW
C = 80
NCHUNK = EPW // C
ZR = 128
RPT = NP // NS

BLK = 512
NBLK = NP // BLK


def _sc_agg_body(xs_hbm, alive_hbm, src_hbm, dst_hbm, agg_hbm, deg_hbm,
                 srcT, dstT, db0, db1, rows0, rows1, vals0, vals1,
                 agg_sh, deg_sh, sg0, sg1, sv0, sv1, ssc0, ssc1, svs0, svs1):
    cid = lax.axis_index("c")
    sid = lax.axis_index("s")
    rows = [rows0, rows1]
    vals = [vals0, vals1]
    db = [db0, db1]
    sg = [sg0, sg1]
    sv = [sv0, sv1]
    ssc = [ssc0, ssc1]
    svs = [svs0, svs1]

    wid = sid * NC + cid
    pltpu.sync_copy(src_hbm.at[pl.ds(wid * EPW, EPW)], srcT)
    pltpu.sync_copy(dst_hbm.at[pl.ds(wid * EPW, EPW)], dstT)

    def zrow(r, carry):
        for j in range(F // 16):
            rows0[r, pl.ds(j * 16, 16)] = jnp.zeros((16,), jnp.float32)
        return carry
    lax.fori_loop(0, C, zrow, 0)
    for j in range(C // 16):
        vals0[pl.ds(j * 16, 16)] = jnp.zeros((16,), jnp.float32)

    r0 = sid * RPT
    for bkt in range(RPT // C):
        pltpu.sync_copy(rows0, agg_sh.at[pl.ds(r0 + bkt * C, C)])
        pltpu.sync_copy(vals0, deg_sh.at[pl.ds(r0 + bkt * C, C)])
    plsc.subcore_barrier()

    def gather(i, b):
        sl = srcT.at[pl.ds(i * C, C)]
        for j in range(C // 16):
            db[b][pl.ds(j * 16, 16)] = dstT[pl.ds(i * C + j * 16, 16)]
        pltpu.async_copy(xs_hbm.at[sl], rows[b], sg[b])
        pltpu.async_copy(alive_hbm.at[sl], vals[b], sv[b])

    def wait_gather(b):
        pltpu.make_async_copy(xs_hbm.at[pl.ds(0, C)], rows[b], sg[b]).wait()
        pltpu.make_async_copy(alive_hbm.at[pl.ds(0, C)], vals[b], sv[b]).wait()

    def scatter(b):
        pltpu.async_copy(rows[b], agg_sh.at[db[b]], ssc[b], add=True)
        pltpu.async_copy(vals[b], deg_sh.at[db[b]], svs[b], add=True)

    def wait_scatter(b):
        pltpu.make_async_copy(xs_hbm.at[pl.ds(0, C)], rows[b], ssc[b]).wait()
        pltpu.make_async_copy(alive_hbm.at[pl.ds(0, C)], vals[b], svs[b]).wait()

    gather(0, 0)
    gather(1, 1)
    wait_gather(0)
    scatter(0)

    def pair(g, carry):
        for b0 in range(2):
            i = 2 * g + 1 + b0
            b = (1 + b0) & 1
            wait_scatter(1 - b)
            gather(i + 1, 1 - b)
            wait_gather(b)
            scatter(b)
        return carry
    lax.fori_loop(0, (NCHUNK - 3) // 2, pair, 0)
    wait_scatter(0)
    gather(NCHUNK - 1, 0)
    wait_gather(1)
    scatter(1)
    wait_gather(0)
    scatter(0)
    wait_scatter(1)
    wait_scatter(0)
    plsc.subcore_barrier()

    pltpu.sync_copy(agg_sh.at[pl.ds(r0, RPT)],
                    agg_hbm.at[pl.ds(cid * NP + r0, RPT)])
    pltpu.sync_copy(deg_sh.at[pl.ds(r0, RPT)],
                    deg_hbm.at[pl.ds(cid * NP + r0, RPT)])


@functools.cache
def _get_sc_agg():
    return pl.kernel(
        _sc_agg_body,
        out_type=[
            jax.ShapeDtypeStruct((2 * NP, F), jnp.float32),
            jax.ShapeDtypeStruct((2 * NP,), jnp.float32),
        ],
        mesh=plsc.VectorSubcoreMesh(core_axis_name="c", subcore_axis_name="s",
                                    num_cores=NC, num_subcores=NS),
        scratch_types=[
            pltpu.VMEM((EPW,), jnp.int32),
            pltpu.VMEM((EPW,), jnp.int32),
            pltpu.VMEM((C,), jnp.int32),
            pltpu.VMEM((C,), jnp.int32),
            pltpu.VMEM((C, F), jnp.float32),
            pltpu.VMEM((C, F), jnp.float32),
            pltpu.VMEM((C,), jnp.float32),
            pltpu.VMEM((C,), jnp.float32),
            pltpu.VMEM_SHARED((NP, F), jnp.float32),
            pltpu.VMEM_SHARED((NP,), jnp.float32),
        ] + [pltpu.SemaphoreType.DMA] * 8,
        name="sc_edge_agg",
    )


def _tc1_body(agg_ref, deg_ref, xs_ref, alive_ref,
              wl_ref, wr_ref, b_ref, p_ref, h_ref, s_ref, key_ref):
    a3 = agg_ref[...]
    a = a3[0] + a3[1]
    d3 = deg_ref[...]
    deg = d3[0] + d3[1]
    mean = a / jnp.maximum(deg, 1.0)
    x = xs_ref[...]
    alive = alive_ref[...]
    p = p_ref[...]
    nrm = jnp.sqrt(jnp.sum(p * p)) + 1e-16
    pn = p / nrm
    h = jnp.dot(mean, wl_ref[...], preferred_element_type=jnp.float32)
    h = h + jnp.dot(x, wr_ref[...], preferred_element_type=jnp.float32)
    h = jnp.maximum(h + b_ref[...], 0.0) * alive
    s = jnp.tanh(jnp.dot(h, pn, preferred_element_type=jnp.float32))
    h_ref[...] = h
    s_ref[...] = s
    pn_row = pn.reshape(1, F)
    hk = h + (alive - 1.0) * (jnp.float32(1e4) * pn_row)
    rows = []
    for j in range(BLK // 128):
        hj = hk[j * 128:(j + 1) * 128, :]
        srow = jnp.tanh(lax.dot_general(
            pn_row, hj, (((1,), (1,)), ((), ())),
            preferred_element_type=jnp.float32))
        u = lax.bitcast_convert_type(srow, jnp.uint32)
        rows.append(jnp.where(u & jnp.uint32(0x80000000) != jnp.uint32(0),
                              ~u, u | jnp.uint32(0x80000000)))
    key_ref[...] = jnp.concatenate(rows, axis=0)[None]


_GRP = BLK // 128


_tc1 = pl.pallas_call(
    _tc1_body,
    grid=(NBLK,),
    in_specs=[
        pl.BlockSpec((2, BLK, F), lambda i: (0, i, 0)),
        pl.BlockSpec((2, BLK, 1), lambda i: (0, i, 0)),
        pl.BlockSpec((BLK, F), lambda i: (i, 0)),
        pl.BlockSpec((BLK, 1), lambda i: (i, 0)),
        pl.BlockSpec((F, F), lambda i: (0, 0)),
        pl.BlockSpec((F, F), lambda i: (0, 0)),
        pl.BlockSpec((1, F), lambda i: (0, 0)),
        pl.BlockSpec((F, 1), lambda i: (0, 0)),
    ],
    out_specs=[
        pl.BlockSpec((BLK, F), lambda i: (i, 0)),
        pl.BlockSpec((BLK, 1), lambda i: (i, 0)),
        pl.BlockSpec((1, _GRP, 128), lambda i: (i, 0, 0)),
    ],
    out_shape=[
        jax.ShapeDtypeStruct((NP, F), jnp.float32),
        jax.ShapeDtypeStruct((NP, 1), jnp.float32),
        jax.ShapeDtypeStruct((NBLK, _GRP, 128), jnp.uint32),
    ],
)


def _sel_body(key_ref, keep_ref, *, k_out):
    keys = key_ref[...]
    sh = (NBLK, _GRP, 128)
    idx = (lax.broadcasted_iota(jnp.int32, sh, 0) * (_GRP * 128)
           + lax.broadcasted_iota(jnp.int32, sh, 1) * 128
           + lax.broadcasted_iota(jnp.int32, sh, 2))

    def tbody(j, t):
        bit = jnp.left_shift(jnp.uint32(1), jnp.uint32(31) - j.astype(jnp.uint32))
        cand = t | bit
        cnt = jnp.sum((keys >= cand).astype(jnp.int32))
        return jnp.where(cnt >= k_out, cand, t)
    t = lax.fori_loop(0, 32, tbody, jnp.uint32(0))

    cnt_gt = jnp.sum((keys > t).astype(jnp.int32))
    need = k_out - cnt_gt
    eq = keys == t

    def mbody(j, m):
        cand = m | jnp.left_shift(jnp.int32(1), jnp.int32(13) - j)
        f = jnp.sum((eq & (idx < cand)).astype(jnp.int32))
        return jnp.where(f < need, cand, m)
    m = lax.fori_loop(0, 14, mbody, jnp.int32(0))

    keep = (keys > t) | (eq & (idx <= m) & (need > 0))
    keep_ref[...] = keep.astype(jnp.float32)


def _make_sel(k_out):
    return pl.pallas_call(
        functools.partial(_sel_body, k_out=k_out),
        out_shape=jax.ShapeDtypeStruct((NBLK, _GRP, 128), jnp.float32),
    )


def _pack_body(h_ref, s_ref, keep_ref, xs_ref, g_ref):
    i = pl.program_id(0)
    h = h_ref[...]
    s = s_ref[...]
    k2 = keep_ref[...][0]
    eye = (lax.broadcasted_iota(jnp.int32, (_GRP, _GRP), 0)
           == lax.broadcasted_iota(jnp.int32, (_GRP, _GRP), 1)).astype(jnp.float32)
    kt = lax.dot_general(k2, eye, (((0,), (0,)), ((), ())),
                         preferred_element_type=jnp.float32)
    keep = jnp.concatenate([kt[:, j:j + 1] for j in range(_GRP)], axis=0)
    xn = h * s * keep
    xs_ref[...] = xn
    blk_max = jnp.max(jnp.where(keep > 0.0, xn, jnp.float32(-3.4e38)),
                      axis=0, keepdims=True)
    blk_sum = jnp.sum(xn, axis=0, keepdims=True)

    @pl.when(i == 0)
    def _():
        g_ref[...] = jnp.concatenate(
            [jnp.full((1, F), -3.4e38, jnp.float32),
             jnp.zeros((1, F), jnp.float32)], axis=1)
    cur = g_ref[...]
    g_ref[...] = jnp.concatenate(
        [jnp.maximum(cur[:, :F], blk_max), cur[:, F:] + blk_sum], axis=1)


_pack = pl.pallas_call(
    _pack_body,
    grid=(NBLK,),
    in_specs=[
        pl.BlockSpec((BLK, F), lambda i: (i, 0)),
        pl.BlockSpec((BLK, 1), lambda i: (i, 0)),
        pl.BlockSpec((1, _GRP, 128), lambda i: (i, 0, 0)),
    ],
    out_specs=[
        pl.BlockSpec((BLK, F), lambda i: (i, 0)),
        pl.BlockSpec((1, 2 * F), lambda i: (0, 0)),
    ],
    out_shape=[
        jax.ShapeDtypeStruct((NP, F), jnp.float32),
        jax.ShapeDtypeStruct((1, 2 * F), jnp.float32),
    ],
)


def _mlp_body(g1_ref, g2_ref, g3_ref, w1_ref, b1_ref, w2_ref, b2_ref,
              w3_ref, b3_ref, o_ref, *, k_list):
    def gz(g, k):
        return jnp.concatenate([g[:, :F], g[:, F:] / jnp.float32(k)], axis=1)
    z = (gz(g1_ref[...], k_list[0]) + gz(g2_ref[...], k_list[1])
         + gz(g3_ref[...], k_list[2]))
    z = jnp.maximum(jnp.dot(z, w1_ref[...], preferred_element_type=jnp.float32)
                    + b1_ref[...], 0.0)
    z = jnp.maximum(jnp.dot(z, w2_ref[...], preferred_element_type=jnp.float32)
                    + b2_ref[...], 0.0)
    o_ref[...] = (jnp.dot(z, w3_ref[...], preferred_element_type=jnp.float32)
                  + b3_ref[...])


def _make_mlp(k_list):
    return pl.pallas_call(
        functools.partial(_mlp_body, k_list=k_list),
        out_shape=jax.ShapeDtypeStruct((1, T_OUT), jnp.float32),
    )


_K_SEQ = [8000, 6400, 5120]
_SEL = [_make_sel(k) for k in _K_SEQ]
_MLP = _make_mlp(_K_SEQ)


def kernel(x, edge_index, batch, Wl1, Wr1, b1, p1, Wl2, Wr2, b2, p2,
           Wl3, Wr3, b3, p3, W1, bl1, W2, bl2, W3, bl3):
    src = edge_index[0]
    dst = edge_index[1]
    xs = jnp.zeros((NP, F), jnp.float32).at[:N].set(x)
    alive = jnp.zeros((NP,), jnp.float32).at[:N].set(1.0)

    layer_params = [(Wl1, Wr1, b1, p1), (Wl2, Wr2, b2, p2), (Wl3, Wr3, b3, p3)]
    g_out = []
    for l in range(3):
        Wl, Wr, b, p = layer_params[l]
        agg, deg = _get_sc_agg()(xs, alive, src, dst)
        h, s, key = _tc1(agg.reshape(2, NP, F), deg.reshape(2, NP, 1),
                         xs, alive.reshape(NP, 1), Wl, Wr, b.reshape(1, F),
                         p.reshape(F, 1))
        keep = _SEL[l](key)
        xs, g = _pack(h, s, keep)
        alive = keep.reshape(NP)
        g_out.append(g)

    return _MLP(g_out[0], g_out[1], g_out[2],
                W1, bl1.reshape(1, -1), W2, bl2.reshape(1, -1),
                W3, bl3.reshape(1, -1))


# speedup vs baseline: 28.5381x; 1.1545x over previous
import functools

import jax
import jax.numpy as jnp
from jax import lax
from jax.experimental import pallas as pl
from jax.experimental.pallas import tpu as pltpu
from jax.experimental.pallas import tpu_sc as plsc

N = 10000
NP = 10240
E = 320000
F = 128
T_OUT = 10

NC = 2
NS = 16
NW = NC * NS
EPW = E // NW
C = 80
NCHUNK = EPW // C
ZR = 128
RPT = NP // NS

BLK = 1024
NBLK = NP // BLK


def _sc_agg_body(xs_hbm, alive_hbm, src_hbm, dst_hbm, agg_hbm, deg_hbm,
                 srcT, dstT, db0, db1, rows0, rows1, vals0, vals1,
                 agg_sh, deg_sh, sg0, sg1, sv0, sv1, ssc0, ssc1, svs0, svs1):
    cid = lax.axis_index("c")
    sid = lax.axis_index("s")
    rows = [rows0, rows1]
    vals = [vals0, vals1]
    db = [db0, db1]
    sg = [sg0, sg1]
    sv = [sv0, sv1]
    ssc = [ssc0, ssc1]
    svs = [svs0, svs1]

    wid = sid * NC + cid
    pltpu.sync_copy(src_hbm.at[pl.ds(wid * EPW, EPW)], srcT)
    pltpu.sync_copy(dst_hbm.at[pl.ds(wid * EPW, EPW)], dstT)

    def zrow(r, carry):
        for j in range(F // 16):
            rows0[r, pl.ds(j * 16, 16)] = jnp.zeros((16,), jnp.float32)
        return carry
    lax.fori_loop(0, C, zrow, 0)
    for j in range(C // 16):
        vals0[pl.ds(j * 16, 16)] = jnp.zeros((16,), jnp.float32)

    r0 = sid * RPT
    for bkt in range(RPT // C):
        pltpu.sync_copy(rows0, agg_sh.at[pl.ds(r0 + bkt * C, C)])
        pltpu.sync_copy(vals0, deg_sh.at[pl.ds(r0 + bkt * C, C)])
    plsc.subcore_barrier()

    def gather(i, b):
        sl = srcT.at[pl.ds(i * C, C)]
        for j in range(C // 16):
            db[b][pl.ds(j * 16, 16)] = dstT[pl.ds(i * C + j * 16, 16)]
        pltpu.async_copy(xs_hbm.at[sl], rows[b], sg[b])
        pltpu.async_copy(alive_hbm.at[sl], vals[b], sv[b])

    def wait_gather(b):
        pltpu.make_async_copy(xs_hbm.at[pl.ds(0, C)], rows[b], sg[b]).wait()
        pltpu.make_async_copy(alive_hbm.at[pl.ds(0, C)], vals[b], sv[b]).wait()

    def scatter(b):
        pltpu.async_copy(rows[b], agg_sh.at[db[b]], ssc[b], add=True)
        pltpu.async_copy(vals[b], deg_sh.at[db[b]], svs[b], add=True)

    def wait_scatter(b):
        pltpu.make_async_copy(xs_hbm.at[pl.ds(0, C)], rows[b], ssc[b]).wait()
        pltpu.make_async_copy(alive_hbm.at[pl.ds(0, C)], vals[b], svs[b]).wait()

    gather(0, 0)
    gather(1, 1)
    wait_gather(0)
    scatter(0)

    def pair(g, carry):
        for b0 in range(2):
            i = 2 * g + 1 + b0
            b = (1 + b0) & 1
            wait_scatter(1 - b)
            gather(i + 1, 1 - b)
            wait_gather(b)
            scatter(b)
        return carry
    lax.fori_loop(0, (NCHUNK - 3) // 2, pair, 0)
    wait_scatter(0)
    gather(NCHUNK - 1, 0)
    wait_gather(1)
    scatter(1)
    wait_gather(0)
    scatter(0)
    wait_scatter(1)
    wait_scatter(0)
    plsc.subcore_barrier()

    pltpu.sync_copy(agg_sh.at[pl.ds(r0, RPT)],
                    agg_hbm.at[cid, pl.ds(r0, RPT)])
    pltpu.sync_copy(deg_sh.at[pl.ds(r0, RPT)],
                    deg_hbm.at[cid, pl.ds(r0, RPT)])


@functools.cache
def _get_sc_agg():
    return pl.kernel(
        _sc_agg_body,
        out_type=[
            jax.ShapeDtypeStruct((2, NP, F), jnp.float32),
            jax.ShapeDtypeStruct((2, NP), jnp.float32),
        ],
        mesh=plsc.VectorSubcoreMesh(core_axis_name="c", subcore_axis_name="s",
                                    num_cores=NC, num_subcores=NS),
        scratch_types=[
            pltpu.VMEM((EPW,), jnp.int32),
            pltpu.VMEM((EPW,), jnp.int32),
            pltpu.VMEM((C,), jnp.int32),
            pltpu.VMEM((C,), jnp.int32),
            pltpu.VMEM((C, F), jnp.float32),
            pltpu.VMEM((C, F), jnp.float32),
            pltpu.VMEM((C,), jnp.float32),
            pltpu.VMEM((C,), jnp.float32),
            pltpu.VMEM_SHARED((NP, F), jnp.float32),
            pltpu.VMEM_SHARED((NP,), jnp.float32),
        ] + [pltpu.SemaphoreType.DMA] * 8,
        name="sc_edge_agg",
    )


_GRP = BLK // 128


def _eye(n):
    return (lax.broadcasted_iota(jnp.int32, (n, n), 0)
            == lax.broadcasted_iota(jnp.int32, (n, n), 1)).astype(jnp.float32)


def _tc1_body(agg_ref, deg_ref, xs_ref, alive_ref,
              wl_ref, wr_ref, b_ref, p_ref, h_ref, s_ref, key_ref):
    a3 = agg_ref[...]
    a = a3[0] + a3[1]
    d2 = deg_ref[...]
    dt = lax.dot_general(d2, _eye(2), (((0,), (0,)), ((), ())),
                         preferred_element_type=jnp.float32)
    deg = dt[:, 0:1] + dt[:, 1:2]
    a2 = alive_ref[...]
    at = lax.dot_general(a2, _eye(_GRP), (((0,), (0,)), ((), ())),
                         preferred_element_type=jnp.float32)
    alive = jnp.concatenate([at[:, j:j + 1] for j in range(_GRP)], axis=0)
    mean = a / jnp.maximum(deg, 1.0)
    x = xs_ref[...]
    p = p_ref[...]
    nrm = jnp.sqrt(jnp.sum(p * p)) + 1e-16
    pn = p / nrm
    h = jnp.dot(mean, wl_ref[...], preferred_element_type=jnp.float32)
    h = h + jnp.dot(x, wr_ref[...], preferred_element_type=jnp.float32)
    h = jnp.maximum(h + b_ref[...], 0.0) * alive
    s = jnp.tanh(jnp.dot(h, pn, preferred_element_type=jnp.float32))
    h_ref[...] = h
    s_ref[...] = s
    pn_row = pn.reshape(1, F)
    hk = h + (alive - 1.0) * (jnp.float32(1e4) * pn_row)
    rows = []
    for j in range(_GRP):
        hj = hk[j * 128:(j + 1) * 128, :]
        srow = jnp.tanh(lax.dot_general(
            pn_row, hj, (((1,), (1,)), ((), ())),
            preferred_element_type=jnp.float32))
        u = lax.bitcast_convert_type(srow, jnp.uint32)
        rows.append(jnp.where(u & jnp.uint32(0x80000000) != jnp.uint32(0),
                              ~u, u | jnp.uint32(0x80000000)))
    key_ref[...] = jnp.concatenate(rows, axis=0)


_tc1 = pl.pallas_call(
    _tc1_body,
    grid=(NBLK,),
    in_specs=[
        pl.BlockSpec((2, BLK, F), lambda i: (0, i, 0)),
        pl.BlockSpec((2, BLK), lambda i: (0, i)),
        pl.BlockSpec((BLK, F), lambda i: (i, 0)),
        pl.BlockSpec((_GRP, 128), lambda i: (i, 0)),
        pl.BlockSpec((F, F), lambda i: (0, 0)),
        pl.BlockSpec((F, F), lambda i: (0, 0)),
        pl.BlockSpec((1, F), lambda i: (0, 0)),
        pl.BlockSpec((F, 1), lambda i: (0, 0)),
    ],
    out_specs=[
        pl.BlockSpec((BLK, F), lambda i: (i, 0)),
        pl.BlockSpec((BLK, 1), lambda i: (i, 0)),
        pl.BlockSpec((_GRP, 128), lambda i: (i, 0)),
    ],
    out_shape=[
        jax.ShapeDtypeStruct((NP, F), jnp.float32),
        jax.ShapeDtypeStruct((NP, 1), jnp.float32),
        jax.ShapeDtypeStruct((NP // 128, 128), jnp.uint32),
    ],
)


def _select(keys, k_out):
    sh = (NP // 128, 128)
    idx = (lax.broadcasted_iota(jnp.int32, sh, 0) * 128
           + lax.broadcasted_iota(jnp.int32, sh, 1))

    def tbody(j, t):
        bit = jnp.left_shift(jnp.uint32(1), jnp.uint32(31) - j.astype(jnp.uint32))
        cand = t | bit
        cnt = jnp.sum((keys >= cand).astype(jnp.int32))
        return jnp.where(cnt >= k_out, cand, t)
    t = lax.fori_loop(0, 32, tbody, jnp.uint32(0))

    cnt_gt = jnp.sum((keys > t).astype(jnp.int32))
    need = k_out - cnt_gt
    eq = keys == t

    def mbody(j, m):
        cand = m | jnp.left_shift(jnp.int32(1), jnp.int32(13) - j)
        f = jnp.sum((eq & (idx < cand)).astype(jnp.int32))
        return jnp.where(f < need, cand, m)
    m = lax.fori_loop(0, 14, mbody, jnp.int32(0))

    keep = (keys > t) | (eq & (idx <= m) & (need > 0))
    return keep.astype(jnp.float32)


def _pack_body(h_ref, s_ref, key_ref, xs_ref, g_ref, keep_ref, keep_s, *, k_out):
    i = pl.program_id(0)

    @pl.when(i == 0)
    def _():
        keep0 = _select(key_ref[...], k_out)
        keep_s[...] = keep0
        keep_ref[...] = keep0

    h = h_ref[...]
    s = s_ref[...]
    k2 = keep_s[pl.ds(i * _GRP, _GRP), :]
    kt = lax.dot_general(k2, _eye(_GRP), (((0,), (0,)), ((), ())),
                         preferred_element_type=jnp.float32)
    keep = jnp.concatenate([kt[:, j:j + 1] for j in range(_GRP)], axis=0)
    xn = h * s * keep
    xs_ref[...] = xn
    blk_max = jnp.max(jnp.where(keep > 0.0, xn, jnp.float32(-3.4e38)),
                      axis=0, keepdims=True)
    blk_sum = jnp.sum(xn, axis=0, keepdims=True)

    @pl.when(i == 0)
    def _():
        g_ref[...] = jnp.concatenate(
            [jnp.full((1, F), -3.4e38, jnp.float32),
             jnp.zeros((1, F), jnp.float32)], axis=1)
    cur = g_ref[...]
    g_ref[...] = jnp.concatenate(
        [jnp.maximum(cur[:, :F], blk_max), cur[:, F:] + blk_sum], axis=1)


def _make_pack(k_out):
    return pl.pallas_call(
        functools.partial(_pack_body, k_out=k_out),
        grid=(NBLK,),
        in_specs=[
            pl.BlockSpec((BLK, F), lambda i: (i, 0)),
            pl.BlockSpec((BLK, 1), lambda i: (i, 0)),
            pl.BlockSpec((NP // 128, 128), lambda i: (0, 0)),
        ],
        out_specs=[
            pl.BlockSpec((BLK, F), lambda i: (i, 0)),
            pl.BlockSpec((1, 2 * F), lambda i: (0, 0)),
            pl.BlockSpec((NP // 128, 128), lambda i: (0, 0)),
        ],
        out_shape=[
            jax.ShapeDtypeStruct((NP, F), jnp.float32),
            jax.ShapeDtypeStruct((1, 2 * F), jnp.float32),
            jax.ShapeDtypeStruct((NP // 128, 128), jnp.float32),
        ],
        scratch_shapes=[pltpu.VMEM((NP // 128, 128), jnp.float32)],
    )


def _mlp_body(g1_ref, g2_ref, g3_ref, w1_ref, b1_ref, w2_ref, b2_ref,
              w3_ref, b3_ref, o_ref, *, k_list):
    def gz(g, k):
        return jnp.concatenate([g[:, :F], g[:, F:] / jnp.float32(k)], axis=1)
    z = (gz(g1_ref[...], k_list[0]) + gz(g2_ref[...], k_list[1])
         + gz(g3_ref[...], k_list[2]))
    z = jnp.maximum(jnp.dot(z, w1_ref[...], preferred_element_type=jnp.float32)
                    + b1_ref[...], 0.0)
    z = jnp.maximum(jnp.dot(z, w2_ref[...], preferred_element_type=jnp.float32)
                    + b2_ref[...], 0.0)
    o_ref[...] = (jnp.dot(z, w3_ref[...], preferred_element_type=jnp.float32)
                  + b3_ref[...])


def _make_mlp(k_list):
    return pl.pallas_call(
        functools.partial(_mlp_body, k_list=k_list),
        out_shape=jax.ShapeDtypeStruct((1, T_OUT), jnp.float32),
    )


_K_SEQ = [8000, 6400, 5120]
_PACK = [_make_pack(k) for k in _K_SEQ]
_MLP = _make_mlp(_K_SEQ)


def kernel(x, edge_index, batch, Wl1, Wr1, b1, p1, Wl2, Wr2, b2, p2,
           Wl3, Wr3, b3, p3, W1, bl1, W2, bl2, W3, bl3):
    src = edge_index[0]
    dst = edge_index[1]
    xs = jnp.zeros((NP, F), jnp.float32).at[:N].set(x)
    alive = jnp.zeros((NP,), jnp.float32).at[:N].set(1.0)
    alive2 = alive.reshape(NP // 128, 128)

    layer_params = [(Wl1, Wr1, b1, p1), (Wl2, Wr2, b2, p2), (Wl3, Wr3, b3, p3)]
    g_out = []
    for l in range(3):
        Wl, Wr, b, p = layer_params[l]
        agg, deg = _get_sc_agg()(xs, alive, src, dst)
        h, s, key = _tc1(agg, deg, xs, alive2,
                         Wl, Wr, b.reshape(1, F), p.reshape(F, 1))
        xs, g, keep = _PACK[l](h, s, key)
        alive = keep.reshape(NP)
        alive2 = keep
        g_out.append(g)

    return _MLP(g_out[0], g_out[1], g_out[2],
                W1, bl1.reshape(1, -1), W2, bl2.reshape(1, -1),
                W3, bl3.reshape(1, -1))
